# baseline (speedup 1.0000x reference)
import jax
import jax.numpy as jnp
from jax.experimental import pallas as pl

B, S, D = 4, 8192, 4096


def _body(x_ref, o_ref):
    o_ref[...] = x_ref[...]


def kernel(hidden_state):
    flat = hidden_state.reshape(B, S * D)
    out = pl.pallas_call(
        _body,
        grid=(1,),
        in_specs=[pl.BlockSpec((B, D), lambda i: (0, 0))],
        out_specs=pl.BlockSpec((B, D), lambda i: (0, 0)),
        out_shape=jax.ShapeDtypeStruct((B, D), jnp.float32),
    )(flat)
    return out.reshape(B, 1, D)


# speedup vs baseline: 219.4052x; 219.4052x over previous
import jax
import jax.numpy as jnp
from jax.experimental import pallas as pl

B, S, D = 4, 8192, 4096


def _body(x_ref, o_ref):
    o_ref[...] = x_ref[:, 0:1, :]


def kernel(hidden_state):
    return pl.pallas_call(
        _body,
        grid=(1,),
        in_specs=[pl.BlockSpec((B, 8, D), lambda i: (0, 0, 0))],
        out_specs=pl.BlockSpec((B, 1, D), lambda i: (0, 0, 0)),
        out_shape=jax.ShapeDtypeStruct((B, 1, D), jnp.float32),
    )(hidden_state)
